# baseline (speedup 1.0000x reference)
import functools

import jax
import jax.numpy as jnp
from jax import lax
from jax.experimental import pallas as pl
from jax.experimental.pallas import tpu as pltpu
from jax.experimental.pallas import tpu_sc as plsc

B, L = 16384, 50
N = B * L
D = 64
PIN = 128

N_CORES, N_SUBCORES = 2, 16
NW = N_CORES * N_SUBCORES
CHUNK = 128
NBUF = 4
NACC = 8

ROWS_TC = 6400

NSLICE = 4
NS = N // NSLICE
GRID_S = NS // ROWS_TC


def _sc_gather_mse(ids_flat, table, proj_s, off_rows, ns):
    mesh = plsc.VectorSubcoreMesh(core_axis_name="c", subcore_axis_name="s")
    rows_per_w = ns // NW
    n_chunks = rows_per_w // CHUNK
    steady = ((n_chunks - NBUF) // NBUF) * NBUF

    @functools.partial(
        pl.kernel,
        out_type=jax.ShapeDtypeStruct((NW * 16,), jnp.float32),
        mesh=mesh,
        scratch_types=[
            pltpu.VMEM((rows_per_w,), jnp.int32),
            [pltpu.VMEM((CHUNK // 2, D), jnp.float32) for _ in range(NBUF)],
            [pltpu.VMEM((CHUNK // 2, D), jnp.float32) for _ in range(NBUF)],
            [pltpu.VMEM((CHUNK // 2, 2 * D), jnp.float32)
             for _ in range(NBUF)],
            pltpu.VMEM((NACC, 16), jnp.float32),
            [pltpu.SemaphoreType.DMA for _ in range(NBUF)],
            [pltpu.SemaphoreType.DMA for _ in range(NBUF)],
        ],
        compiler_params=pltpu.CompilerParams(use_tc_tiling_on_sc=False),
    )
    def k(ids_hbm, table_hbm, proj_hbm, out_hbm, idx_v, rows_e, rows_o,
          p_v, acc_v, gsem, psem):
        wid = lax.axis_index("s") * N_CORES + lax.axis_index("c")
        base = wid * rows_per_w

        pltpu.sync_copy(ids_hbm.at[pl.ds(off_rows + base, rows_per_w)],
                        idx_v)
        for a in range(NACC):
            acc_v[a, :] = jnp.zeros((16,), jnp.float32)

        def start(j, s):
            blk = j // (ROWS_TC // CHUNK)
            off = (blk * ROWS_TC
                   + (j % (ROWS_TC // CHUNK)) * (CHUNK // 2))
            pltpu.async_copy(
                table_hbm.at[idx_v.at[pl.ds(off, CHUNK // 2)]],
                rows_e[s], gsem[s])
            pltpu.async_copy(
                table_hbm.at[idx_v.at[pl.ds(off + ROWS_TC // 2,
                                            CHUNK // 2)]],
                rows_o[s], gsem[s])
            prow = (base + j * CHUNK) // 2
            pltpu.async_copy(proj_hbm.at[pl.ds(prow, CHUNK // 2)],
                             p_v[s], psem[s])

        def wait(s):
            pltpu.make_async_copy(table_hbm.at[idx_v.at[pl.ds(0, CHUNK // 2)]],
                                  rows_e[s], gsem[s]).wait()
            pltpu.make_async_copy(table_hbm.at[idx_v.at[pl.ds(0, CHUNK // 2)]],
                                  rows_o[s], gsem[s]).wait()
            pltpu.make_async_copy(proj_hbm.at[pl.ds(0, CHUNK // 2)],
                                  p_v[s], psem[s]).wait()

        def compute(s):
            @pl.loop(0, CHUNK // 2)
            def _(r):
                for c in range(8):
                    src = rows_e[s] if c < 4 else rows_o[s]
                    dv = (p_v[s][r, pl.ds(16 * c, 16)]
                          - src[r, pl.ds(16 * (c % 4), 16)])
                    acc_v[c, :] += dv * dv

        for s in range(NBUF):
            start(s, s)

        @pl.loop(0, steady, step=NBUF)
        def _(i):
            for s in range(NBUF):
                j = i + s
                wait(s)
                compute(s)
                start(j + NBUF, s)

        for j in range(steady, n_chunks - NBUF):
            s = j % NBUF
            wait(s)
            compute(s)
            start(j + NBUF, s)

        for j in range(n_chunks - NBUF, n_chunks):
            s = j % NBUF
            wait(s)
            compute(s)

        for a in range(1, NACC):
            acc_v[0, :] += acc_v[a, :]
        pltpu.sync_copy(acc_v.at[0], out_hbm.at[pl.ds(wid * 16, 16)])

    return k(ids_flat, table, proj_s)


def _tc_project(x2d, W1, b1, W2, b2, grid_off):
    def body(x_ref, w1_ref, b1_ref, w2_ref, b2_ref, out_ref):
        h = jnp.dot(x_ref[...], w1_ref[...],
                    preferred_element_type=jnp.float32) + b1_ref[...]
        h = 0.5 * h * (1.0 + lax.erf(h * jnp.float32(0.7071067811865476)))
        p = jnp.dot(h, w2_ref[...],
                    preferred_element_type=jnp.float32) + b2_ref[...]
        out_ref[...] = jnp.concatenate(
            [p[0:ROWS_TC // 2], p[ROWS_TC // 2:ROWS_TC]], axis=1)

    return pl.pallas_call(
        body,
        grid=(GRID_S,),
        in_specs=[
            pl.BlockSpec((ROWS_TC, PIN), lambda i: (i + grid_off, 0)),
            pl.BlockSpec((PIN, D), lambda i: (0, 0)),
            pl.BlockSpec((1, D), lambda i: (0, 0)),
            pl.BlockSpec((D, D), lambda i: (0, 0)),
            pl.BlockSpec((1, D), lambda i: (0, 0)),
        ],
        out_specs=pl.BlockSpec((ROWS_TC // 2, 2 * D), lambda i: (i, 0)),
        out_shape=jax.ShapeDtypeStruct((NS // 2, 2 * D), jnp.float32),
    )(x2d, W1, b1, W2, b2)


def kernel(cluster_ids, dense_embeddings, table, W1, b1, W2, b2):
    ids_flat = cluster_ids.transpose(1, 0).reshape(N)
    x2d = dense_embeddings.transpose(1, 0, 2).reshape(N, PIN)
    b1r, b2r = b1.reshape(1, D), b2.reshape(1, D)
    projs = [_tc_project(x2d, W1, b1r, W2, b2r, sl * GRID_S)
             for sl in range(NSLICE)]
    total = jnp.float32(0)
    for sl in range(NSLICE):
        part = _sc_gather_mse(ids_flat, table, projs[sl], sl * NS, NS)
        total = total + jnp.sum(part)
    return total / jnp.float32(N * D)


# speedup vs baseline: 1.9190x; 1.9190x over previous
import functools

import jax
import jax.numpy as jnp
from jax import lax
from jax.experimental import pallas as pl
from jax.experimental.pallas import tpu as pltpu
from jax.experimental.pallas import tpu_sc as plsc

B, L = 16384, 50
N = B * L
D = 64
PIN = 128

N_CORES, N_SUBCORES = 2, 16
NW = N_CORES * N_SUBCORES
CHUNK = 128
NBUF = 4

ROWS_TC = 6400

NSLICE = 4
NS = N // NSLICE
GRID_S = NS // ROWS_TC


def _sc_gather(ids_flat, table, off_rows, ns):
    mesh = plsc.VectorSubcoreMesh(core_axis_name="c", subcore_axis_name="s")
    rows_per_w = ns // NW
    n_chunks = rows_per_w // CHUNK
    steady = ((n_chunks - NBUF) // NBUF) * NBUF

    @functools.partial(
        pl.kernel,
        out_type=jax.ShapeDtypeStruct((ns // 2, 2 * D), jnp.float32),
        mesh=mesh,
        scratch_types=[
            pltpu.VMEM((rows_per_w,), jnp.int32),
            [pltpu.VMEM((CHUNK // 2, D), jnp.float32) for _ in range(NBUF)],
            [pltpu.VMEM((CHUNK // 2, D), jnp.float32) for _ in range(NBUF)],
            [pltpu.SemaphoreType.DMA for _ in range(NBUF)],
            [pltpu.SemaphoreType.DMA for _ in range(NBUF)],
        ],
        compiler_params=pltpu.CompilerParams(use_tc_tiling_on_sc=False),
    )
    def k(ids_hbm, table_hbm, out_hbm, idx_v, rows_e, rows_o, gsem, osem):
        wid = lax.axis_index("s") * N_CORES + lax.axis_index("c")
        base = wid * rows_per_w

        pltpu.sync_copy(ids_hbm.at[pl.ds(off_rows + base, rows_per_w)],
                        idx_v)

        def gather(j, s):
            blk = j // (ROWS_TC // CHUNK)
            off = (blk * ROWS_TC
                   + (j % (ROWS_TC // CHUNK)) * (CHUNK // 2))
            pltpu.async_copy(
                table_hbm.at[idx_v.at[pl.ds(off, CHUNK // 2)]],
                rows_e[s], gsem[s])
            pltpu.async_copy(
                table_hbm.at[idx_v.at[pl.ds(off + ROWS_TC // 2,
                                            CHUNK // 2)]],
                rows_o[s], gsem[s])

        def wait_gather(s):
            pltpu.make_async_copy(table_hbm.at[idx_v.at[pl.ds(0, CHUNK // 2)]],
                                  rows_e[s], gsem[s]).wait()
            pltpu.make_async_copy(table_hbm.at[idx_v.at[pl.ds(0, CHUNK // 2)]],
                                  rows_o[s], gsem[s]).wait()

        def writeback(j, s):
            prow = (base + j * CHUNK) // 2
            pltpu.async_copy(rows_e[s],
                             out_hbm.at[pl.ds(prow, CHUNK // 2),
                                        pl.ds(0, D)], osem[s])
            pltpu.async_copy(rows_o[s],
                             out_hbm.at[pl.ds(prow, CHUNK // 2),
                                        pl.ds(D, D)], osem[s])

        def wait_writeback(s):
            pltpu.make_async_copy(rows_e[s],
                                  out_hbm.at[pl.ds(base // 2, CHUNK // 2),
                                             pl.ds(0, D)], osem[s]).wait()
            pltpu.make_async_copy(rows_o[s],
                                  out_hbm.at[pl.ds(base // 2, CHUNK // 2),
                                             pl.ds(D, D)], osem[s]).wait()

        for s in range(NBUF):
            gather(s, s)

        @pl.loop(0, steady, step=NBUF)
        def _(i):
            for s in range(NBUF):
                j = i + s
                wait_gather(s)
                writeback(j, s)
                wait_writeback(s)
                gather(j + NBUF, s)

        for j in range(steady, n_chunks - NBUF):
            s = j % NBUF
            wait_gather(s)
            writeback(j, s)
            wait_writeback(s)
            gather(j + NBUF, s)

        for j in range(n_chunks - NBUF, n_chunks):
            s = j % NBUF
            wait_gather(s)
            writeback(j, s)
            wait_writeback(s)

    return k(ids_flat, table)


def _tc_mse_sum(x2d, lookup_s, W1, b1, W2, b2, grid_off):
    def body(x_ref, l_ref, w1_ref, b1_ref, w2_ref, b2_ref, out_ref):
        h = jnp.dot(x_ref[...], w1_ref[...],
                    preferred_element_type=jnp.float32) + b1_ref[...]
        h = 0.5 * h * (1.0 + lax.erf(h * jnp.float32(0.7071067811865476)))
        p = jnp.dot(h, w2_ref[...],
                    preferred_element_type=jnp.float32) + b2_ref[...]
        lk = l_ref[...]
        d1 = lk[:, 0:D] - p[0:ROWS_TC // 2]
        d2 = lk[:, D:2 * D] - p[ROWS_TC // 2:ROWS_TC]
        s = jnp.sum(d1 * d1) + jnp.sum(d2 * d2)

        @pl.when(pl.program_id(0) == 0)
        def _():
            out_ref[...] = jnp.zeros((1, 1), jnp.float32)

        out_ref[...] += jnp.reshape(s, (1, 1))

    return pl.pallas_call(
        body,
        grid=(GRID_S,),
        in_specs=[
            pl.BlockSpec((ROWS_TC, PIN), lambda i: (i + grid_off, 0)),
            pl.BlockSpec((ROWS_TC // 2, 2 * D), lambda i: (i, 0)),
            pl.BlockSpec((PIN, D), lambda i: (0, 0)),
            pl.BlockSpec((1, D), lambda i: (0, 0)),
            pl.BlockSpec((D, D), lambda i: (0, 0)),
            pl.BlockSpec((1, D), lambda i: (0, 0)),
        ],
        out_specs=pl.BlockSpec((1, 1), lambda i: (0, 0)),
        out_shape=jax.ShapeDtypeStruct((1, 1), jnp.float32),
    )(x2d, lookup_s, W1, b1, W2, b2)


def kernel(cluster_ids, dense_embeddings, table, W1, b1, W2, b2):
    ids_flat = cluster_ids.transpose(1, 0).reshape(N)
    x2d = dense_embeddings.transpose(1, 0, 2).reshape(N, PIN)
    b1r, b2r = b1.reshape(1, D), b2.reshape(1, D)
    lookups = [_sc_gather(ids_flat, table, sl * NS, NS)
               for sl in range(NSLICE)]
    total = jnp.float32(0)
    for sl in range(NSLICE):
        part = _tc_mse_sum(x2d, lookups[sl], W1, b1r, W2, b2r,
                           sl * GRID_S)
        total = total + part[0, 0]
    return total / jnp.float32(N * D)
